# Initial kernel scaffold; baseline (speedup 1.0000x reference)
import jax
import jax.numpy as jnp
from jax.experimental import pallas as pl


def kernel(x, internal_edge_index, all_edge_index, params):
    raise NotImplementedError("write your pallas kernel here")


# speedup vs baseline: 5.1971x; 5.1971x over previous
import functools

import jax
import jax.numpy as jnp
from jax import lax
from jax.experimental import pallas as pl
from jax.experimental.pallas import tpu as pltpu
from jax.experimental.pallas import tpu_sc as plsc

N = 10000
D = 128
E = 320000
NC = 2
NS = 16
NW = NC * NS
EPW = E // NW
K = 128
CH = 81
EPAD = CH * K
NPAD = 10240
RPS = NPAD // NS
TRASH = N
DW = 16
GRP = 1000
NG = N // GRP
EPS = 1e-5
F32 = jnp.float32
HI = lax.Precision.HIGHEST

@functools.cache
def _mesh():
    return plsc.VectorSubcoreMesh(core_axis_name="c", subcore_axis_name="s",
                                  num_cores=NC, num_subcores=NS)


def _prep_edges(ei):
    pad = EPAD - EPW
    r = jnp.pad(ei[0].reshape(NW, EPW), ((0, 0), (0, pad)), constant_values=0)
    c = jnp.pad(ei[1].reshape(NW, EPW), ((0, 0), (0, pad)),
                constant_values=TRASH)
    return r.reshape(NW, CH, K), c.reshape(NW, CH, K)


DEGB = 9


def _sc_deg_body(colp2, out, col_v, gbuf, acc, sem):
    cid = lax.axis_index("c")
    sid = lax.axis_index("s")
    wid = cid * NS + sid

    pltpu.sync_copy(colp2.at[wid], col_v)

    zero16 = jnp.zeros((16,), F32)
    one16 = jnp.ones((16,), F32)

    def zrow(r, _):
        for cc in range(D // 16):
            gbuf[r, pl.ds(cc * 16, 16)] = zero16
        return 0
    lax.fori_loop(0, K, zrow, 0)
    for i in range(RPS // K):
        pltpu.sync_copy(gbuf, acc.at[pl.ds(sid * RPS + i * K, K)])

    def orow(r, _):
        for cc in range(D // 16):
            gbuf[r, pl.ds(cc * 16, 16)] = one16
        return 0
    lax.fori_loop(0, K, orow, 0)
    plsc.subcore_barrier()

    def chunkb(t, _):
        for u in range(DEGB):
            pltpu.async_copy(gbuf, acc.at[col_v.at[t * DEGB + u]], sem,
                             add=True)
        for u in range(DEGB):
            pltpu.make_async_copy(gbuf, acc.at[col_v.at[t * DEGB + u]],
                                  sem).wait()
        return 0
    lax.fori_loop(0, (2 * CH) // DEGB, chunkb, 0)

    plsc.subcore_barrier()
    rows = pl.ds(sid * RPS, RPS)
    pltpu.sync_copy(acc.at[rows], out.at[cid, rows])


def _deg_call(colp_i, colp_a):
    colp2 = jnp.concatenate([colp_i.reshape(NS, 2 * CH, K),
                             colp_a.reshape(NS, 2 * CH, K)], axis=0)
    k = pl.kernel(
        _sc_deg_body,
        out_type=jax.ShapeDtypeStruct((NC, NPAD, D), F32),
        mesh=_mesh(),
        scratch_types=[
            pltpu.VMEM((2 * CH, K), jnp.int32),
            pltpu.VMEM((K, D), F32),
            pltpu.VMEM_SHARED((NPAD, D), F32),
            pltpu.SemaphoreType.DMA,
        ],
    )
    return k(colp2)


def _sc_scatter_body(hp, rowp, colp, out, row_v, col_v, gbuf_a, acc,
                     sem_g, sem_s):
    cid = lax.axis_index("c")
    sid = lax.axis_index("s")
    wid = cid * NS + sid

    pltpu.sync_copy(rowp.at[wid], row_v)
    pltpu.sync_copy(colp.at[wid], col_v)

    zero16 = jnp.zeros((16,), F32)

    def zrow(r, _):
        for cc in range(D // 16):
            gbuf_a[r, pl.ds(cc * 16, 16)] = zero16
        return 0
    lax.fori_loop(0, K, zrow, 0)
    for i in range(RPS // K):
        pltpu.sync_copy(gbuf_a, acc.at[pl.ds(sid * RPS + i * K, K)])
    plsc.subcore_barrier()

    def chunk(j, _):
        pltpu.async_copy(hp.at[row_v.at[j]], gbuf_a, sem_g).wait()
        pltpu.async_copy(gbuf_a, acc.at[col_v.at[j]], sem_s, add=True).wait()
        return 0
    lax.fori_loop(0, CH, chunk, 0)

    plsc.subcore_barrier()
    rows = pl.ds(sid * RPS, RPS)
    pltpu.sync_copy(acc.at[rows], out.at[cid, rows])


def _scatter_call(hp, rowp, colp):
    k = pl.kernel(
        _sc_scatter_body,
        out_type=jax.ShapeDtypeStruct((NC, NPAD, D), F32),
        mesh=_mesh(),
        scratch_types=[
            pltpu.VMEM((CH, K), jnp.int32),
            pltpu.VMEM((CH, K), jnp.int32),
            pltpu.VMEM((K, D), F32),
            pltpu.VMEM_SHARED((NPAD, D), F32),
            pltpu.SemaphoreType.DMA,
            pltpu.SemaphoreType.DMA,
        ],
    )
    return k(hp, rowp, colp)


def _tc_dis_body(deg_ref, disi_ref, disa_ref):
    d = deg_ref[...]
    disi_ref[...] = lax.rsqrt(d[0, :, 0:1] + 1.0)
    disa_ref[...] = lax.rsqrt(d[1, :, 0:1] + 1.0)


def _dis_call(deg):
    nb = 8
    blk = NPAD // nb
    return pl.pallas_call(
        _tc_dis_body,
        grid=(nb,),
        in_specs=[pl.BlockSpec((NC, blk, D), lambda i: (0, i, 0))],
        out_specs=[pl.BlockSpec((blk, 1), lambda i: (i, 0)),
                   pl.BlockSpec((blk, 1), lambda i: (i, 0))],
        out_shape=[jax.ShapeDtypeStruct((NPAD, 1), F32),
                   jax.ShapeDtypeStruct((NPAD, 1), F32)],
    )(deg)


def _tc_in_body(x_ref, w0t_ref, b0_ref, w1t_ref, b1_ref, dis_ref,
                h_ref, hp_ref):
    xi = jnp.dot(x_ref[...], w0t_ref[...], precision=HI) + b0_ref[...]
    h = jnp.dot(xi, w1t_ref[...], precision=HI) + b1_ref[...]
    h_ref[...] = h
    hp_ref[...] = dis_ref[...] * h


def _in_call(x, w0t, b0, w1t, b1, dis):
    full = pl.BlockSpec((1, D), lambda i: (0, 0))
    wspec = pl.BlockSpec((D, D), lambda i: (0, 0))
    rows = pl.BlockSpec((GRP, D), lambda i: (i, 0))
    return pl.pallas_call(
        _tc_in_body,
        grid=(NG,),
        in_specs=[rows, wspec, full, wspec, full,
                  pl.BlockSpec((GRP, 1), lambda i: (i, 0))],
        out_specs=[rows, rows],
        out_shape=[jax.ShapeDtypeStruct((N, D), F32),
                   jax.ShapeDtypeStruct((N, D), F32)],
    )(x, w0t, b0, w1t, b1, dis)


def _leaky(x):
    return jnp.where(x >= 0, x, 0.1 * x)


def _relu(x):
    return jnp.where(x >= 0, x, 0.0)


def _make_mid_body(act):
    def body(s_ref, h_ref, dis_ref, g_ref, bt_ref, wt_ref, b_ref, disn_ref,
             hn_ref, hpn_ref, gv, ssum, s2sum):
        p = pl.program_id(0)
        i = pl.program_id(1)

        @pl.when(p == 0)
        def _phase0():
            dis = dis_ref[...]
            g = dis * (s_ref[0] + s_ref[1]) + (dis * dis) * h_ref[...]
            gv[pl.ds(i * GRP, GRP), :] = g

            @pl.when(i == 0)
            def _init():
                ssum[...] = jnp.zeros((1, D), F32)
                s2sum[...] = jnp.zeros((1, D), F32)
            ssum[...] += jnp.sum(g, axis=0, keepdims=True)
            s2sum[...] += jnp.sum(g * g, axis=0, keepdims=True)

        @pl.when(p == 1)
        def _phase1():
            mu = ssum[...] * (1.0 / N)
            var = s2sum[...] * (1.0 / N) - mu * mu
            inv = lax.rsqrt(var + EPS) * g_ref[...]
            g = gv[pl.ds(i * GRP, GRP), :]
            a = act((g - mu) * inv + bt_ref[...])
            hn = jnp.dot(a, wt_ref[...], precision=HI) + b_ref[...]
            hn_ref[...] = hn
            hpn_ref[...] = disn_ref[...] * hn
    return body


def _mid_specs():
    srows = pl.BlockSpec((NC, GRP, D), lambda p, i: (0, i * (1 - p), 0))
    rows = pl.BlockSpec((GRP, D), lambda p, i: (i * (1 - p), 0))
    rows1 = pl.BlockSpec((GRP, 1), lambda p, i: (i, 0))
    full = pl.BlockSpec((1, D), lambda p, i: (0, 0))
    wspec = pl.BlockSpec((D, D), lambda p, i: (0, 0))
    orows = pl.BlockSpec((GRP, D), lambda p, i: (i, 0))
    scratch = [pltpu.VMEM((N, D), F32), pltpu.VMEM((1, D), F32),
               pltpu.VMEM((1, D), F32)]
    return srows, rows, rows1, full, wspec, orows, scratch


_MID_RELU = _make_mid_body(_relu)
_MID_LEAKY = _make_mid_body(_leaky)


def _mid_call(body, s, h, dis, g, bt, wt, b, disn):
    srows, rows, rows1, full, wspec, orows, scratch = _mid_specs()
    return pl.pallas_call(
        body,
        grid=(2, NG),
        in_specs=[srows, rows, rows1, full, full, wspec, full, rows1],
        out_specs=[orows, orows],
        out_shape=[jax.ShapeDtypeStruct((N, D), F32),
                   jax.ShapeDtypeStruct((N, D), F32)],
        scratch_shapes=scratch,
    )(s, h, dis, g, bt, wt, b, disn)


def _concat_body(s_ref, h_ref, dis_ref, g_ref, bt_ref, x_ref, wxt_ref,
                 wht_ref, bm_ref, w1t_ref, b1_ref, disn_ref,
                 hn_ref, hpn_ref, gv, ssum, s2sum):
    p = pl.program_id(0)
    i = pl.program_id(1)

    @pl.when(p == 0)
    def _phase0():
        dis = dis_ref[...]
        g = dis * (s_ref[0] + s_ref[1]) + (dis * dis) * h_ref[...]
        gv[pl.ds(i * GRP, GRP), :] = g

        @pl.when(i == 0)
        def _init():
            ssum[...] = jnp.zeros((1, D), F32)
            s2sum[...] = jnp.zeros((1, D), F32)
        ssum[...] += jnp.sum(g, axis=0, keepdims=True)
        s2sum[...] += jnp.sum(g * g, axis=0, keepdims=True)

    @pl.when(p == 1)
    def _phase1():
        mu = ssum[...] * (1.0 / N)
        var = s2sum[...] * (1.0 / N) - mu * mu
        inv = lax.rsqrt(var + EPS) * g_ref[...]
        g = gv[pl.ds(i * GRP, GRP), :]
        a = _relu((g - mu) * inv + bt_ref[...])
        xc = (jnp.dot(x_ref[...], wxt_ref[...], precision=HI)
              + jnp.dot(a, wht_ref[...], precision=HI) + bm_ref[...])
        hn = jnp.dot(xc, w1t_ref[...], precision=HI) + b1_ref[...]
        hn_ref[...] = hn
        hpn_ref[...] = disn_ref[...] * hn


def _concat_call(s, h, dis, g, bt, x, wxt, wht, bm, w1t, b1, disn):
    srows, rows, rows1, full, wspec, orows, scratch = _mid_specs()
    xrows = pl.BlockSpec((GRP, D), lambda p, i: (i * p, 0))
    return pl.pallas_call(
        _concat_body,
        grid=(2, NG),
        in_specs=[srows, rows, rows1, full, full, xrows, wspec, wspec, full,
                  wspec, full, rows1],
        out_specs=[orows, orows],
        out_shape=[jax.ShapeDtypeStruct((N, D), F32),
                   jax.ShapeDtypeStruct((N, D), F32)],
        scratch_shapes=scratch,
    )(s, h, dis, g, bt, x, wxt, wht, bm, w1t, b1, disn)


def _pow_or_id(x, pw):
    powed = jnp.where(x > 0.0,
                      jnp.exp(pw * jnp.log(jnp.maximum(x, 1e-30))), 0.0)
    return jnp.where(pw == 1.0, x, powed)


def _head_body(s_ref, h_ref, dis_ref, g_ref, bt_ref, pw_ref, wgt_ref, bg_ref,
               out_ref, yp_ref, gv, ssum, s2sum, pooled):
    p = pl.program_id(0)
    i = pl.program_id(1)

    @pl.when(p == 0)
    def _phase0():
        dis = dis_ref[...]
        g = dis * (s_ref[0] + s_ref[1]) + (dis * dis) * h_ref[...]
        gv[pl.ds(i * GRP, GRP), :] = g

        @pl.when(i == 0)
        def _init():
            ssum[...] = jnp.zeros((1, D), F32)
            s2sum[...] = jnp.zeros((1, D), F32)
        ssum[...] += jnp.sum(g, axis=0, keepdims=True)
        s2sum[...] += jnp.sum(g * g, axis=0, keepdims=True)

    @pl.when(p == 1)
    def _phase1():
        pw = pw_ref[0, 0]
        mu = ssum[...] * (1.0 / N)
        var = s2sum[...] * (1.0 / N) - mu * mu
        inv = lax.rsqrt(var + EPS) * g_ref[...]
        g = gv[pl.ds(i * GRP, GRP), :]
        a = _leaky((g - mu) * inv + bt_ref[...])
        xp = _pow_or_id(jnp.clip(a, 0.0, 100.0), pw)
        pooled[pl.ds(i, 1), :] = jnp.sum(xp, axis=0, keepdims=True) * (1.0 / GRP)

        @pl.when(i == NG - 1)
        def _final():
            yq = _pow_or_id(jnp.clip(pooled[...], 0.0, 100.0), 1.0 / pw)
            logits = jnp.dot(yq, wgt_ref[...], precision=HI) + bg_ref[...]
            out_ref[...] = logits
            m = jnp.max(logits, axis=1, keepdims=True)
            io = lax.broadcasted_iota(jnp.int32, (NG, 10), 1)
            yp_ref[...] = jnp.min(
                jnp.where(logits == m, io, jnp.int32(2 ** 30)),
                axis=1, keepdims=True)


def _head_call(s, h, dis, g, bt, pw, wgt, bg):
    srows, rows, rows1, full, wspec, orows, scratch = _mid_specs()
    one = pl.BlockSpec((1, 1), lambda p, i: (0, 0))
    return pl.pallas_call(
        _head_body,
        grid=(2, NG),
        in_specs=[srows, rows, rows1, full, full, one,
                  pl.BlockSpec((D, 10), lambda p, i: (0, 0)),
                  pl.BlockSpec((1, 10), lambda p, i: (0, 0))],
        out_specs=[pl.BlockSpec((NG, 10), lambda p, i: (0, 0)),
                   pl.BlockSpec((NG, 1), lambda p, i: (0, 0))],
        out_shape=[jax.ShapeDtypeStruct((NG, 10), F32),
                   jax.ShapeDtypeStruct((NG, 1), jnp.int32)],
        scratch_shapes=scratch + [pltpu.VMEM((NG, D), F32)],
    )(s, h, dis, g, bt, pw, wgt, bg)


def kernel(x, internal_edge_index, all_edge_index, params):
    p = params
    rowp_i, colp_i = _prep_edges(internal_edge_index)
    rowp_a, colp_a = _prep_edges(all_edge_index)

    deg = _deg_call(colp_i, colp_a)
    dis_i, dis_a = _dis_call(deg)

    def r1(v):
        return v.reshape(1, -1)

    h1, hp1 = _in_call(x, p['W_it'].T, r1(p['b_it']),
                       p['iW1'].T, r1(p['ib1']), dis_i[:N])

    s1 = _scatter_call(hp1, rowp_i, colp_i)
    h2, hp2 = _mid_call(_MID_RELU, s1, h1, dis_i[:N], r1(p['ig1']),
                        r1(p['ibt1']), p['iW2'].T, r1(p['ib2']), dis_i[:N])

    s2 = _scatter_call(hp2, rowp_i, colp_i)
    h3, hp3 = _mid_call(_MID_RELU, s2, h2, dis_i[:N], r1(p['ig2']),
                        r1(p['ibt2']), p['iW3'].T, r1(p['ib3']), dis_i[:N])

    s3 = _scatter_call(hp3, rowp_i, colp_i)
    wmt = p['W_m'].T
    h4, hp4 = _concat_call(s3, h3, dis_i[:N], r1(p['ig3']), r1(p['ibt3']),
                           x, wmt[:D], wmt[D:], r1(p['b_m']),
                           p['gW1'].T, r1(p['gb1']), dis_a[:N])

    s4 = _scatter_call(hp4, rowp_a, colp_a)
    h5, hp5 = _mid_call(_MID_LEAKY, s4, h4, dis_a[:N], r1(p['gg1']),
                        r1(p['gbt1']), p['gW2'].T, r1(p['gb2']), dis_a[:N])

    s5 = _scatter_call(hp5, rowp_a, colp_a)
    h6, hp6 = _mid_call(_MID_LEAKY, s5, h5, dis_a[:N], r1(p['gg2']),
                        r1(p['gbt2']), p['gW3'].T, r1(p['gb3']), dis_a[:N])

    s6 = _scatter_call(hp6, rowp_a, colp_a)
    out, ypi = _head_call(s6, h6, dis_a[:N], r1(p['gg3']), r1(p['gbt3']),
                          p['p_pow'].reshape(1, 1), p['Wg'].T, r1(p['bg']))

    return out, ypi.reshape(NG)
